# Initial kernel scaffold; baseline (speedup 1.0000x reference)
import jax
import jax.numpy as jnp
from jax.experimental import pallas as pl


def kernel(x, edge_index, W1, b1, W_mu, b_mu, W_logstd, b_logstd):
    raise NotImplementedError("write your pallas kernel here")


# speedup vs baseline: 10.7046x; 10.7046x over previous
import functools

import jax
import jax.numpy as jnp
from jax import lax
from jax.experimental import pallas as pl
from jax.experimental.pallas import tpu as pltpu
from jax.experimental.pallas import tpu_sc as plsc

_N = 10000
_E = 320000
_D = 128
_NC = 2
_NS = 16
_NW = _NC * _NS
_NP = 10240
_RT = _NP // _NS
_CHUNK = 128
_KC = 80
_EP = _NW * _KC * _CHUNK
_BR = 512


def _sc_mesh():
    return plsc.VectorSubcoreMesh(core_axis_name="c", subcore_axis_name="s")


@functools.partial(
    pl.kernel,
    out_type=jax.ShapeDtypeStruct((_NC, _NP), jnp.float32),
    mesh=_sc_mesh(),
    scratch_types=[
        pltpu.VMEM((_KC, _CHUNK), jnp.int32),
        pltpu.VMEM((_CHUNK,), jnp.float32),
        pltpu.VMEM((_RT,), jnp.float32),
        pltpu.VMEM_SHARED((_NP,), jnp.float32),
    ],
)
def _deg_kernel(dst_hbm, out_hbm, dstv, ones_v, zb, shist):
    cid = lax.axis_index("c")
    sid = lax.axis_index("s")
    wid = cid * _NS + sid
    pltpu.sync_copy(dst_hbm.at[pl.ds(wid * _KC, _KC)], dstv)

    def fill_ones(i, carry):
        ones_v[pl.ds(i * 16, 16)] = jnp.full((16,), 1.0, jnp.float32)
        return carry

    lax.fori_loop(0, _CHUNK // 16, fill_ones, 0)

    def fill_zeros(i, carry):
        zb[pl.ds(i * 16, 16)] = jnp.zeros((16,), jnp.float32)
        return carry

    lax.fori_loop(0, _RT // 16, fill_zeros, 0)
    pltpu.sync_copy(zb, shist.at[pl.ds(sid * _RT, _RT)])
    plsc.subcore_barrier()

    def chunk(j, carry):
        pltpu.sync_copy(ones_v, shist.at[dstv.at[j]], add=True)
        return carry

    lax.fori_loop(0, _KC, chunk, 0)
    plsc.subcore_barrier()
    pltpu.sync_copy(shist.at[pl.ds(sid * _RT, _RT)],
                    out_hbm.at[cid, pl.ds(sid * _RT, _RT)])


@functools.partial(
    pl.kernel,
    out_type=jax.ShapeDtypeStruct((_NC, _NP, _D), jnp.float32),
    mesh=_sc_mesh(),
    scratch_types=[
        pltpu.VMEM((_KC, _CHUNK), jnp.int32),
        pltpu.VMEM((_KC, _CHUNK), jnp.int32),
        pltpu.VMEM((_CHUNK, _D), jnp.float32),
        pltpu.VMEM_SHARED((_NP, _D), jnp.float32),
        pltpu.SemaphoreType.DMA,
    ],
)
def _agg_kernel(table_hbm, src_hbm, dst_hbm, out_hbm,
                srcv, dstv, rows, acc, sem):
    cid = lax.axis_index("c")
    sid = lax.axis_index("s")
    wid = cid * _NS + sid
    pltpu.sync_copy(src_hbm.at[pl.ds(wid * _KC, _KC)], srcv)
    pltpu.sync_copy(dst_hbm.at[pl.ds(wid * _KC, _KC)], dstv)

    def zrow(i, carry):
        def zlane(k, c2):
            rows[i, pl.ds(k * 16, 16)] = jnp.zeros((16,), jnp.float32)
            return c2
        return lax.fori_loop(0, _D // 16, zlane, carry)

    lax.fori_loop(0, _CHUNK, zrow, 0)

    def zcp(m, carry):
        pltpu.sync_copy(rows, acc.at[pl.ds(sid * _RT + m * _CHUNK, _CHUNK)])
        return carry

    lax.fori_loop(0, _RT // _CHUNK, zcp, 0)
    plsc.subcore_barrier()

    def chunk(j, carry):
        pltpu.async_copy(table_hbm.at[srcv.at[j]], rows, sem).wait()
        pltpu.sync_copy(rows, acc.at[dstv.at[j]], add=True)
        return carry

    lax.fori_loop(0, _KC, chunk, 0)
    plsc.subcore_barrier()
    pltpu.sync_copy(acc.at[pl.ds(sid * _RT, _RT)],
                    out_hbm.at[cid, pl.ds(sid * _RT, _RT)])


def _dinv_of(degr):
    return lax.rsqrt(degr[0, :] + degr[1, :] + 1.0)


def _mm_scale_body(xr, wr, degr, ur):
    dinv = _dinv_of(degr)
    ur[...] = jnp.dot(xr[...], wr[...],
                      preferred_element_type=jnp.float32) * dinv[:, None]


def _ew_body(accr, ur, degr, br, outr):
    dinv = _dinv_of(degr)[:, None]
    g = dinv * (accr[0] + accr[1] + ur[...])
    outr[...] = jnp.maximum(g + br[...], 0.0) * dinv


def _fin_body(accr, ur, degr, wr, br, outr):
    dinv = _dinv_of(degr)[:, None]
    g = dinv * (accr[0] + accr[1] + ur[...])
    outr[...] = jnp.dot(g, wr[...],
                        preferred_element_type=jnp.float32) + br[...]


def _row_spec(cols):
    return pl.BlockSpec((_BR, cols), lambda i: (i, 0))


def _mm_scale(x_p, W, deg):
    return pl.pallas_call(
        _mm_scale_body,
        grid=(_NP // _BR,),
        in_specs=[
            _row_spec(_D),
            pl.BlockSpec((_D, _D), lambda i: (0, 0)),
            pl.BlockSpec((_NC, _BR), lambda i: (0, i)),
        ],
        out_specs=_row_spec(_D),
        out_shape=jax.ShapeDtypeStruct((_NP, _D), jnp.float32),
    )(x_p, W, deg)


def _ew(acc, u, deg, b):
    return pl.pallas_call(
        _ew_body,
        grid=(_NP // _BR,),
        in_specs=[
            pl.BlockSpec((_NC, _BR, _D), lambda i: (0, i, 0)),
            _row_spec(_D),
            pl.BlockSpec((_NC, _BR), lambda i: (0, i)),
            pl.BlockSpec((1, _D), lambda i: (0, 0)),
        ],
        out_specs=_row_spec(_D),
        out_shape=jax.ShapeDtypeStruct((_NP, _D), jnp.float32),
    )(acc, u, deg, b)


def _fin(acc, u, deg, Wcat, bcat):
    return pl.pallas_call(
        _fin_body,
        grid=(_NP // _BR,),
        in_specs=[
            pl.BlockSpec((_NC, _BR, _D), lambda i: (0, i, 0)),
            _row_spec(_D),
            pl.BlockSpec((_NC, _BR), lambda i: (0, i)),
            pl.BlockSpec((_D, 2 * _D), lambda i: (0, 0)),
            pl.BlockSpec((1, 2 * _D), lambda i: (0, 0)),
        ],
        out_specs=_row_spec(2 * _D),
        out_shape=jax.ShapeDtypeStruct((_NP, 2 * _D), jnp.float32),
    )(acc, u, deg, Wcat, bcat)


def kernel(x, edge_index, W1, b1, W_mu, b_mu, W_logstd, b_logstd):
    src = edge_index[0].astype(jnp.int32)
    dst = edge_index[1].astype(jnp.int32)
    pad_e = _EP - _E
    padv = jnp.full((pad_e,), _N, jnp.int32)
    src_p = jnp.concatenate([src, padv]).reshape(_EP // _CHUNK, _CHUNK)
    dst_p = jnp.concatenate([dst, padv]).reshape(_EP // _CHUNK, _CHUNK)
    x_p = jnp.pad(x, ((0, _NP - _N), (0, 0)))

    deg = _deg_kernel(dst_p)
    u1 = _mm_scale(x_p, W1, deg)
    acc1 = _agg_kernel(u1, src_p, dst_p)
    u2 = _ew(acc1, u1, deg, b1.reshape(1, _D))
    acc2 = _agg_kernel(u2, src_p, dst_p)
    Wcat = jnp.concatenate([W_mu, W_logstd], axis=1)
    bcat = jnp.concatenate([b_mu, b_logstd]).reshape(1, 2 * _D)
    out = _fin(acc2, u2, deg, Wcat, bcat)
    return out[:_N, :_D], out[:_N, _D:]


# speedup vs baseline: 13.3462x; 1.2468x over previous
import functools

import jax
import jax.numpy as jnp
from jax import lax
from jax.experimental import pallas as pl
from jax.experimental.pallas import tpu as pltpu
from jax.experimental.pallas import tpu_sc as plsc

_N = 10000
_E = 320000
_D = 128
_NC = 2
_NS = 16
_NW = _NC * _NS
_NP = 10240
_RT = _NP // _NS

_CHUNK = 128
_KC = 80
_EP = _NW * _KC * _CHUNK

_KD = 80
_CD = 128
_EPD = _NW * _KD * _CD

_BR = 512


def _sc_mesh():
    return plsc.VectorSubcoreMesh(core_axis_name="c", subcore_axis_name="s")


@functools.partial(
    pl.kernel,
    out_type=jax.ShapeDtypeStruct((_NC, _NP), jnp.float32),
    mesh=_sc_mesh(),
    scratch_types=[
        pltpu.VMEM((_KD, _CD), jnp.int32),
        pltpu.VMEM((_CD,), jnp.float32),
        pltpu.VMEM((_RT,), jnp.float32),
        pltpu.VMEM_SHARED((_NP,), jnp.float32),
    ],
)
def _deg_kernel(dst_hbm, out_hbm, dstv, ones_v, zb, shist):
    cid = lax.axis_index("c")
    sid = lax.axis_index("s")
    wid = cid * _NS + sid
    pltpu.sync_copy(dst_hbm.at[pl.ds(wid * _KD, _KD)], dstv)

    def fill_ones(i, carry):
        ones_v[pl.ds(i * 16, 16)] = jnp.full((16,), 1.0, jnp.float32)
        return carry

    lax.fori_loop(0, _CD // 16, fill_ones, 0)

    def fill_zeros(i, carry):
        zb[pl.ds(i * 16, 16)] = jnp.zeros((16,), jnp.float32)
        return carry

    lax.fori_loop(0, _RT // 16, fill_zeros, 0)
    pltpu.sync_copy(zb, shist.at[pl.ds(sid * _RT, _RT)])
    plsc.subcore_barrier()

    def chunk(j, carry):
        pltpu.sync_copy(ones_v, shist.at[dstv.at[j]], add=True)
        return carry

    lax.fori_loop(0, _KD, chunk, 0)
    plsc.subcore_barrier()
    pltpu.sync_copy(shist.at[pl.ds(sid * _RT, _RT)],
                    out_hbm.at[cid, pl.ds(sid * _RT, _RT)])


@functools.partial(
    pl.kernel,
    out_type=jax.ShapeDtypeStruct((_NC, _NP, _D), jnp.float32),
    mesh=_sc_mesh(),
    scratch_types=[
        pltpu.VMEM((_KC, _CHUNK), jnp.int32),
        pltpu.VMEM((_CHUNK,), jnp.int32),
        pltpu.VMEM((_CHUNK,), jnp.int32),
        pltpu.VMEM((_CHUNK,), jnp.int32),
        pltpu.VMEM((_CHUNK,), jnp.int32),
        pltpu.VMEM((_CHUNK, _D), jnp.float32),
        pltpu.VMEM((_CHUNK, _D), jnp.float32),
        pltpu.VMEM_SHARED((_NP, _D), jnp.float32),
        pltpu.SemaphoreType.DMA,
        pltpu.SemaphoreType.DMA,
    ],
)
def _agg_kernel(table_hbm, ed_hbm, out_hbm,
                edv, src0, dst0, src1, dst1, rows0, rows1, acc, sem0, sem1):
    cid = lax.axis_index("c")
    sid = lax.axis_index("s")
    wid = cid * _NS + sid
    pltpu.sync_copy(ed_hbm.at[wid], edv)

    def zrow(i, carry):
        def zlane(k, c2):
            rows0[i, pl.ds(k * 16, 16)] = jnp.zeros((16,), jnp.float32)
            return c2
        return lax.fori_loop(0, _D // 16, zlane, carry)

    lax.fori_loop(0, _CHUNK, zrow, 0)

    def zcp(m, carry):
        pltpu.sync_copy(rows0, acc.at[pl.ds(sid * _RT + m * _CHUNK, _CHUNK)])
        return carry

    lax.fori_loop(0, _RT // _CHUNK, zcp, 0)
    plsc.subcore_barrier()

    def unpack(j, sbuf, dbuf):
        def lane(k, carry):
            v = edv[j, pl.ds(k * 16, 16)]
            sbuf[pl.ds(k * 16, 16)] = lax.shift_right_logical(v, 16)
            dbuf[pl.ds(k * 16, 16)] = lax.bitwise_and(v, 0xFFFF)
            return carry
        lax.fori_loop(0, _CHUNK // 16, lane, 0)

    def gather(sbuf, rows, sem):
        pltpu.async_copy(table_hbm.at[sbuf], rows, sem)

    def gwait(rows, sem):
        pltpu.make_async_copy(table_hbm.at[pl.ds(0, _CHUNK)], rows, sem).wait()

    unpack(0, src0, dst0)
    gather(src0, rows0, sem0)

    def pair(i, carry):
        j0 = 2 * i
        unpack(j0 + 1, src1, dst1)
        gather(src1, rows1, sem1)
        gwait(rows0, sem0)
        pltpu.sync_copy(rows0, acc.at[dst0], add=True)

        @pl.when(i < _KC // 2 - 1)
        def _():
            unpack(j0 + 2, src0, dst0)
            gather(src0, rows0, sem0)

        gwait(rows1, sem1)
        pltpu.sync_copy(rows1, acc.at[dst1], add=True)
        return carry

    lax.fori_loop(0, _KC // 2, pair, 0)
    plsc.subcore_barrier()
    pltpu.sync_copy(acc.at[pl.ds(sid * _RT, _RT)],
                    out_hbm.at[cid, pl.ds(sid * _RT, _RT)])


def _dinv_of(degr):
    return lax.rsqrt(degr[0, :] + degr[1, :] + 1.0)


def _mm_scale_body(xr, wr, degr, ur):
    dinv = _dinv_of(degr)
    ur[...] = jnp.dot(xr[...], wr[...],
                      preferred_element_type=jnp.float32) * dinv[:, None]


def _ew_body(accr, ur, degr, br, outr):
    dinv = _dinv_of(degr)[:, None]
    g = dinv * (accr[0] + accr[1] + ur[...])
    outr[...] = jnp.maximum(g + br[...], 0.0) * dinv


def _fin_body(accr, ur, degr, wr, br, outr):
    dinv = _dinv_of(degr)[:, None]
    g = dinv * (accr[0] + accr[1] + ur[...])
    outr[...] = jnp.dot(g, wr[...],
                        preferred_element_type=jnp.float32) + br[...]


def _row_spec(cols):
    return pl.BlockSpec((_BR, cols), lambda i: (i, 0))


def _mm_scale(x_p, W, deg):
    return pl.pallas_call(
        _mm_scale_body,
        grid=(_NP // _BR,),
        in_specs=[
            _row_spec(_D),
            pl.BlockSpec((_D, _D), lambda i: (0, 0)),
            pl.BlockSpec((_NC, _BR), lambda i: (0, i)),
        ],
        out_specs=_row_spec(_D),
        out_shape=jax.ShapeDtypeStruct((_NP, _D), jnp.float32),
    )(x_p, W, deg)


def _ew(acc, u, deg, b):
    return pl.pallas_call(
        _ew_body,
        grid=(_NP // _BR,),
        in_specs=[
            pl.BlockSpec((_NC, _BR, _D), lambda i: (0, i, 0)),
            _row_spec(_D),
            pl.BlockSpec((_NC, _BR), lambda i: (0, i)),
            pl.BlockSpec((1, _D), lambda i: (0, 0)),
        ],
        out_specs=_row_spec(_D),
        out_shape=jax.ShapeDtypeStruct((_NP, _D), jnp.float32),
    )(acc, u, deg, b)


def _fin(acc, u, deg, Wcat, bcat):
    return pl.pallas_call(
        _fin_body,
        grid=(_NP // _BR,),
        in_specs=[
            pl.BlockSpec((_NC, _BR, _D), lambda i: (0, i, 0)),
            _row_spec(_D),
            pl.BlockSpec((_NC, _BR), lambda i: (0, i)),
            pl.BlockSpec((_D, 2 * _D), lambda i: (0, 0)),
            pl.BlockSpec((1, 2 * _D), lambda i: (0, 0)),
        ],
        out_specs=_row_spec(2 * _D),
        out_shape=jax.ShapeDtypeStruct((_NP, 2 * _D), jnp.float32),
    )(acc, u, deg, Wcat, bcat)


def kernel(x, edge_index, W1, b1, W_mu, b_mu, W_logstd, b_logstd):
    src = edge_index[0].astype(jnp.int32)
    dst = edge_index[1].astype(jnp.int32)
    pad_a = jnp.full((_EP - _E,), _N * 65536 + _N, jnp.int32)
    packed = src * 65536 + dst
    ed_a = jnp.concatenate([packed, pad_a]).reshape(_NW, _KC, _CHUNK)
    pad_d = jnp.full((_EPD - _E,), _N, jnp.int32)
    dst_d = jnp.concatenate([dst, pad_d]).reshape(_EPD // _CD, _CD)
    x_p = jnp.pad(x, ((0, _NP - _N), (0, 0)))

    deg = _deg_kernel(dst_d)
    u1 = _mm_scale(x_p, W1, deg)
    acc1 = _agg_kernel(u1, ed_a)
    u2 = _ew(acc1, u1, deg, b1.reshape(1, _D))
    acc2 = _agg_kernel(u2, ed_a)
    Wcat = jnp.concatenate([W_mu, W_logstd], axis=1)
    bcat = jnp.concatenate([b_mu, b_logstd]).reshape(1, 2 * _D)
    out = _fin(acc2, u2, deg, Wcat, bcat)
    return out[:_N, :_D], out[:_N, _D:]


# speedup vs baseline: 21.1293x; 1.5832x over previous
import functools

import jax
import jax.numpy as jnp
from jax import lax
from jax.experimental import pallas as pl
from jax.experimental.pallas import tpu as pltpu
from jax.experimental.pallas import tpu_sc as plsc

_N = 10000
_E = 320000
_D = 128
_NC = 2
_NS = 16
_NW = _NC * _NS
_NP = 10240
_RT = _NP // _NS

_CHUNK = 128
_KC0 = 118
_KC1 = 40
_E0 = _NS * _KC0 * _CHUNK
_E1 = _NS * _KC1 * _CHUNK

_KD = 80
_CD = 128
_EPD = _NW * _KD * _CD

_BR = 512


def _sc_mesh():
    return plsc.VectorSubcoreMesh(core_axis_name="c", subcore_axis_name="s")


@functools.partial(
    pl.kernel,
    out_type=jax.ShapeDtypeStruct((_NC, _NP), jnp.float32),
    mesh=_sc_mesh(),
    scratch_types=[
        pltpu.VMEM((_KD, _CD), jnp.int32),
        pltpu.VMEM((_CD,), jnp.float32),
        pltpu.VMEM((_RT,), jnp.float32),
        pltpu.VMEM_SHARED((_NP,), jnp.float32),
    ],
)
def _deg_kernel(dst_hbm, out_hbm, dstv, ones_v, zb, shist):
    cid = lax.axis_index("c")
    sid = lax.axis_index("s")
    wid = cid * _NS + sid
    pltpu.sync_copy(dst_hbm.at[pl.ds(wid * _KD, _KD)], dstv)

    def fill_ones(i, carry):
        ones_v[pl.ds(i * 16, 16)] = jnp.full((16,), 1.0, jnp.float32)
        return carry

    lax.fori_loop(0, _CD // 16, fill_ones, 0)

    def fill_zeros(i, carry):
        zb[pl.ds(i * 16, 16)] = jnp.zeros((16,), jnp.float32)
        return carry

    lax.fori_loop(0, _RT // 16, fill_zeros, 0)
    pltpu.sync_copy(zb, shist.at[pl.ds(sid * _RT, _RT)])
    plsc.subcore_barrier()

    def chunk(j, carry):
        pltpu.sync_copy(ones_v, shist.at[dstv.at[j]], add=True)
        return carry

    lax.fori_loop(0, _KD, chunk, 0)
    plsc.subcore_barrier()
    pltpu.sync_copy(shist.at[pl.ds(sid * _RT, _RT)],
                    out_hbm.at[cid, pl.ds(sid * _RT, _RT)])


@functools.partial(
    pl.kernel,
    out_type=jax.ShapeDtypeStruct((_NC, _NP, _D), jnp.float32),
    mesh=_sc_mesh(),
    scratch_types=[
        pltpu.VMEM((_KC0, _CHUNK), jnp.int32),
        pltpu.VMEM((_CHUNK,), jnp.int32),
        pltpu.VMEM((_CHUNK,), jnp.int32),
        pltpu.VMEM((_CHUNK,), jnp.int32),
        pltpu.VMEM((_CHUNK,), jnp.int32),
        pltpu.VMEM((_CHUNK, _D), jnp.float32),
        pltpu.VMEM((_CHUNK, _D), jnp.float32),
        pltpu.VMEM_SHARED((_NP, _D), jnp.float32),
        pltpu.SemaphoreType.DMA,
        pltpu.SemaphoreType.DMA,
    ],
)
def _agg_kernel(table_hbm, ed0_hbm, ed1_hbm, out_hbm,
                edv, src0, dst0, src1, dst1, rows0, rows1, acc, sem0, sem1):
    cid = lax.axis_index("c")
    sid = lax.axis_index("s")

    @pl.when(cid == 0)
    def _():
        pltpu.sync_copy(ed0_hbm.at[sid], edv)

    @pl.when(cid == 1)
    def _():
        pltpu.sync_copy(ed1_hbm.at[sid], edv.at[pl.ds(0, _KC1)])

    def zrow(i, carry):
        def zlane(k, c2):
            rows0[i, pl.ds(k * 16, 16)] = jnp.zeros((16,), jnp.float32)
            return c2
        return lax.fori_loop(0, _D // 16, zlane, carry)

    lax.fori_loop(0, _CHUNK, zrow, 0)

    def zcp(m, carry):
        pltpu.sync_copy(rows0, acc.at[pl.ds(sid * _RT + m * _CHUNK, _CHUNK)])
        return carry

    lax.fori_loop(0, _RT // _CHUNK, zcp, 0)
    plsc.subcore_barrier()

    def unpack(j, sbuf, dbuf):
        def lane(k, carry):
            v = edv[j, pl.ds(k * 16, 16)]
            sbuf[pl.ds(k * 16, 16)] = lax.shift_right_logical(v, 16)
            dbuf[pl.ds(k * 16, 16)] = lax.bitwise_and(v, 0xFFFF)
            return carry
        lax.fori_loop(0, _CHUNK // 16, lane, 0)

    def gather(sbuf, rows, sem):
        pltpu.async_copy(table_hbm.at[sbuf], rows, sem)

    def gwait(rows, sem):
        pltpu.make_async_copy(table_hbm.at[pl.ds(0, _CHUNK)], rows, sem).wait()

    unpack(0, src0, dst0)
    gather(src0, rows0, sem0)

    npairs = jnp.where(cid == 0, _KC0 // 2, _KC1 // 2)

    def pair(i, carry):
        j0 = 2 * i
        unpack(j0 + 1, src1, dst1)
        gather(src1, rows1, sem1)
        gwait(rows0, sem0)
        pltpu.sync_copy(rows0, acc.at[dst0], add=True)

        @pl.when(i < npairs - 1)
        def _():
            unpack(j0 + 2, src0, dst0)
            gather(src0, rows0, sem0)

        gwait(rows1, sem1)
        pltpu.sync_copy(rows1, acc.at[dst1], add=True)
        return carry

    lax.fori_loop(0, npairs, pair, 0)
    plsc.subcore_barrier()
    pltpu.sync_copy(acc.at[pl.ds(sid * _RT, _RT)],
                    out_hbm.at[cid, pl.ds(sid * _RT, _RT)])


def _dinv_of(degr):
    return lax.rsqrt(degr[0, :] + degr[1, :] + 1.0)


def _mm_scale_body(xr, wr, degr, ur):
    dinv = _dinv_of(degr)
    ur[...] = jnp.dot(xr[...], wr[...],
                      preferred_element_type=jnp.float32) * dinv[:, None]


def _ew_body(accr, ur, degr, br, outr):
    dinv = _dinv_of(degr)[:, None]
    g = dinv * (accr[0] + accr[1] + ur[...])
    outr[...] = jnp.maximum(g + br[...], 0.0) * dinv


def _fin_body(accr, ur, degr, wr, br, outr):
    dinv = _dinv_of(degr)[:, None]
    g = dinv * (accr[0] + accr[1] + ur[...])
    outr[...] = jnp.dot(g, wr[...],
                        preferred_element_type=jnp.float32) + br[...]


def _row_spec(cols):
    return pl.BlockSpec((_BR, cols), lambda i: (i, 0))


def _mm_scale(x_p, W, deg):
    return pl.pallas_call(
        _mm_scale_body,
        grid=(_NP // _BR,),
        in_specs=[
            _row_spec(_D),
            pl.BlockSpec((_D, _D), lambda i: (0, 0)),
            pl.BlockSpec((_NC, _BR), lambda i: (0, i)),
        ],
        out_specs=_row_spec(_D),
        out_shape=jax.ShapeDtypeStruct((_NP, _D), jnp.float32),
    )(x_p, W, deg)


def _ew(acc, u, deg, b):
    return pl.pallas_call(
        _ew_body,
        grid=(_NP // _BR,),
        in_specs=[
            pl.BlockSpec((_NC, _BR, _D), lambda i: (0, i, 0)),
            _row_spec(_D),
            pl.BlockSpec((_NC, _BR), lambda i: (0, i)),
            pl.BlockSpec((1, _D), lambda i: (0, 0)),
        ],
        out_specs=_row_spec(_D),
        out_shape=jax.ShapeDtypeStruct((_NP, _D), jnp.float32),
    )(acc, u, deg, b)


def _fin(acc, u, deg, Wcat, bcat):
    return pl.pallas_call(
        _fin_body,
        grid=(_NP // _BR,),
        in_specs=[
            pl.BlockSpec((_NC, _BR, _D), lambda i: (0, i, 0)),
            _row_spec(_D),
            pl.BlockSpec((_NC, _BR), lambda i: (0, i)),
            pl.BlockSpec((_D, 2 * _D), lambda i: (0, 0)),
            pl.BlockSpec((1, 2 * _D), lambda i: (0, 0)),
        ],
        out_specs=_row_spec(2 * _D),
        out_shape=jax.ShapeDtypeStruct((_NP, 2 * _D), jnp.float32),
    )(acc, u, deg, Wcat, bcat)


def kernel(x, edge_index, W1, b1, W_mu, b_mu, W_logstd, b_logstd):
    src = edge_index[0].astype(jnp.int32)
    dst = edge_index[1].astype(jnp.int32)
    packed = src * 65536 + dst
    padv = _N * 65536 + _N
    pad_a = jnp.full((_E0 + _E1 - _E,), padv, jnp.int32)
    ed0 = packed[:_E0].reshape(_NS, _KC0, _CHUNK)
    ed1 = jnp.concatenate([packed[_E0:], pad_a]).reshape(_NS, _KC1, _CHUNK)
    pad_d = jnp.full((_EPD - _E,), _N, jnp.int32)
    dst_d = jnp.concatenate([dst, pad_d]).reshape(_EPD // _CD, _CD)
    x_p = jnp.pad(x, ((0, _NP - _N), (0, 0)))

    deg = _deg_kernel(dst_d)
    u1 = _mm_scale(x_p, W1, deg)
    acc1 = _agg_kernel(u1, ed0, ed1)
    u2 = _ew(acc1, u1, deg, b1.reshape(1, _D))
    acc2 = _agg_kernel(u2, ed0, ed1)
    Wcat = jnp.concatenate([W_mu, W_logstd], axis=1)
    bcat = jnp.concatenate([b_mu, b_logstd]).reshape(1, 2 * _D)
    out = _fin(acc2, u2, deg, Wcat, bcat)
    return out[:_N, :_D], out[:_N, _D:]
